# baseline (speedup 1.0000x reference)
import functools

import jax
import jax.numpy as jnp
import numpy as np
from jax import lax
from jax.experimental import pallas as pl
from jax.experimental.pallas import tpu as pltpu
from jax.experimental.pallas import tpu_sc as plsc

B = 16384
L = 20
NUM_CAST = 1000
NUM_CREW = 1000
EMB = 32
HID = 128

NC = 2
NS = 16
LANES = 16
NW = NC * NS
BPW = B // NW

MW = 4


@functools.cache
def _make_pool_kernel(interpret=False):
    mesh = plsc.VectorSubcoreMesh(
        core_axis_name="c", subcore_axis_name="s",
        num_cores=NC, num_subcores=NS)

    @functools.partial(
        pl.kernel,
        out_type=jax.ShapeDtypeStruct((NW, BPW, 2 * EMB), jnp.float32),
        mesh=mesh,
        scratch_types=[
            pltpu.VMEM((NUM_CAST * EMB // 2,), jnp.int32),
            pltpu.VMEM((NUM_CREW * EMB // 2,), jnp.int32),
            pltpu.VMEM((BPW * L,), jnp.int32),
            pltpu.VMEM((BPW * L,), jnp.int32),
            pltpu.VMEM((BPW, 2 * EMB), jnp.float32),
        ],
        compiler_params=pltpu.CompilerParams(needs_layout_passes=False),
        interpret=interpret,
    )
    def pool(cast_tab_hbm, crew_tab_hbm, cidx_hbm, kidx_hbm, out_hbm,
             cast_v, crew_v, cidx_v, kidx_v, out_v):
        wid = lax.axis_index("s") * NC + lax.axis_index("c")
        pltpu.sync_copy(cast_tab_hbm, cast_v)
        pltpu.sync_copy(crew_tab_hbm, crew_v)
        pltpu.sync_copy(cidx_hbm.at[wid], cidx_v)
        pltpu.sync_copy(kidx_hbm.at[wid], kidx_v)

        def row(b, carry):
            ib = b * L
            for idx_v, tab_v, off in ((cidx_v, cast_v, 0),
                                      (kidx_v, crew_v, EMB)):
                iv0 = idx_v[pl.ds(ib, LANES)] * (EMB // 2)
                iv1 = idx_v[pl.ds(ib + L - LANES, LANES)] * (EMB // 2)
                pa = [None, None, None, None]
                pb = [None, None, None, None]
                for l in range(L):
                    if l == 0:
                        r = iv0[0]
                    elif l < LANES:
                        r = iv0[l]
                    else:
                        r = iv1[l - (L - LANES)]
                    rw = plsc.bitcast(tab_v[pl.ds(r, LANES)], jnp.bfloat16)
                    a, c = plsc.unpack(rw, format=plsc.PackFormat.INTERLEAVED,
                                       preferred_element_type=jnp.float32)
                    k = l % 4
                    pa[k] = a if pa[k] is None else pa[k] + a
                    pb[k] = c if pb[k] is None else pb[k] + c
                ea = (pa[0] + pa[1]) + (pa[2] + pa[3])
                eb = (pb[0] + pb[1]) + (pb[2] + pb[3])
                out_v[b, pl.ds(off, LANES)] = ea
                out_v[b, pl.ds(off + LANES, LANES)] = eb
            return carry

        lax.fori_loop(0, BPW, row, 0, unroll=8)
        pltpu.sync_copy(out_v, out_hbm.at[wid])

    return pool


def _mlp_body(x_ref, w1_ref, b1_ref, w2_ref, b2_ref, o_ref):
    for w in range(MW):
        x = x_ref[w]
        h = lax.dot_general(x, w1_ref[...], (((1,), (0,)), ((), ())),
                            preferred_element_type=jnp.float32)
        h = jnp.maximum(h + b1_ref[...][None, :], 0.0)
        o = lax.dot_general(h, w2_ref[...], (((1,), (0,)), ((), ())),
                            preferred_element_type=jnp.float32)
        o_ref[pl.ds(w * BPW, BPW), :] = o + b2_ref[...][None, :]


@functools.cache
def _make_mlp_call(interpret=False):
    return pl.pallas_call(
        _mlp_body,
        grid=(NW // MW,),
        in_specs=[
            pl.BlockSpec((MW, BPW, 2 * EMB), lambda i: (i, 0, 0)),
            pl.BlockSpec((2 * EMB, HID), lambda i: (0, 0)),
            pl.BlockSpec((HID,), lambda i: (0,)),
            pl.BlockSpec((HID, 1), lambda i: (0, 0)),
            pl.BlockSpec((1,), lambda i: (0,)),
        ],
        out_specs=pl.BlockSpec((MW * BPW, 1), lambda i: (i, 0)),
        out_shape=jax.ShapeDtypeStruct((B, 1), jnp.float32),
        interpret=interpret,
    )


def kernel(cast_idx, crew_idx, cast_table, crew_table, W1, b1, W2, b2):
    cidx = cast_idx.astype(jnp.int32).reshape(NW, BPW * L)
    kidx = crew_idx.astype(jnp.int32).reshape(NW, BPW * L)
    def _pack(tab, n):
        t = (tab * (1.0 / L)).astype(jnp.bfloat16)
        t = jnp.stack([t[:, :EMB // 2], t[:, EMB // 2:]], axis=-1)
        return lax.bitcast_convert_type(t, jnp.int32).reshape(-1)

    pooled = _make_pool_kernel()(
        _pack(cast_table, NUM_CAST), _pack(crew_table, NUM_CREW), cidx, kidx)
    return _make_mlp_call()(pooled, W1, b1, W2, b2)


# speedup vs baseline: 1.0011x; 1.0011x over previous
import functools

import jax
import jax.numpy as jnp
import numpy as np
from jax import lax
from jax.experimental import pallas as pl
from jax.experimental.pallas import tpu as pltpu
from jax.experimental.pallas import tpu_sc as plsc

B = 16384
L = 20
NUM_CAST = 1000
NUM_CREW = 1000
EMB = 32
HID = 128

NC = 2
NS = 16
LANES = 16
NW = NC * NS
BPW = B // NW

MW = 4


@functools.cache
def _make_pool_kernel(interpret=False):
    mesh = plsc.VectorSubcoreMesh(
        core_axis_name="c", subcore_axis_name="s",
        num_cores=NC, num_subcores=NS)

    @functools.partial(
        pl.kernel,
        out_type=jax.ShapeDtypeStruct((NW, BPW, 2 * EMB), jnp.float32),
        mesh=mesh,
        scratch_types=[
            pltpu.VMEM((NUM_CAST * EMB // 2,), jnp.int32),
            pltpu.VMEM((NUM_CREW * EMB // 2,), jnp.int32),
            pltpu.VMEM((BPW * L,), jnp.int32),
            pltpu.VMEM((BPW * L,), jnp.int32),
            pltpu.VMEM((BPW, 2 * EMB), jnp.float32),
        ],
        compiler_params=pltpu.CompilerParams(needs_layout_passes=False),
        interpret=interpret,
    )
    def pool(cast_tab_hbm, crew_tab_hbm, cidx_hbm, kidx_hbm, out_hbm,
             cast_v, crew_v, cidx_v, kidx_v, out_v):
        wid = lax.axis_index("s") * NC + lax.axis_index("c")
        pltpu.sync_copy(cast_tab_hbm, cast_v)
        pltpu.sync_copy(crew_tab_hbm, crew_v)
        pltpu.sync_copy(cidx_hbm.at[wid], cidx_v)
        pltpu.sync_copy(kidx_hbm.at[wid], kidx_v)

        @plsc.parallel_loop(0, BPW, unroll=8)
        def row(b):
            ib = b * L
            for idx_v, tab_v, off in ((cidx_v, cast_v, 0),
                                      (kidx_v, crew_v, EMB)):
                iv0 = idx_v[pl.ds(ib, LANES)] * (EMB // 2)
                iv1 = idx_v[pl.ds(ib + L - LANES, LANES)] * (EMB // 2)
                pa = [None, None, None, None]
                pb = [None, None, None, None]
                for l in range(L):
                    if l == 0:
                        r = iv0[0]
                    elif l < LANES:
                        r = iv0[l]
                    else:
                        r = iv1[l - (L - LANES)]
                    rw = plsc.bitcast(tab_v[pl.ds(r, LANES)], jnp.bfloat16)
                    a, c = plsc.unpack(rw, format=plsc.PackFormat.INTERLEAVED,
                                       preferred_element_type=jnp.float32)
                    k = l % 4
                    pa[k] = a if pa[k] is None else pa[k] + a
                    pb[k] = c if pb[k] is None else pb[k] + c
                ea = (pa[0] + pa[1]) + (pa[2] + pa[3])
                eb = (pb[0] + pb[1]) + (pb[2] + pb[3])
                out_v[b, pl.ds(off, LANES)] = ea
                out_v[b, pl.ds(off + LANES, LANES)] = eb

        pltpu.sync_copy(out_v, out_hbm.at[wid])

    return pool


def _mlp_body(x_ref, w1_ref, b1_ref, w2_ref, b2_ref, o_ref):
    for w in range(MW):
        x = x_ref[w]
        h = lax.dot_general(x, w1_ref[...], (((1,), (0,)), ((), ())),
                            preferred_element_type=jnp.float32)
        h = jnp.maximum(h + b1_ref[...][None, :], 0.0)
        o = lax.dot_general(h, w2_ref[...], (((1,), (0,)), ((), ())),
                            preferred_element_type=jnp.float32)
        o_ref[pl.ds(w * BPW, BPW), :] = o + b2_ref[...][None, :]


@functools.cache
def _make_mlp_call(interpret=False):
    return pl.pallas_call(
        _mlp_body,
        grid=(NW // MW,),
        in_specs=[
            pl.BlockSpec((MW, BPW, 2 * EMB), lambda i: (i, 0, 0)),
            pl.BlockSpec((2 * EMB, HID), lambda i: (0, 0)),
            pl.BlockSpec((HID,), lambda i: (0,)),
            pl.BlockSpec((HID, 1), lambda i: (0, 0)),
            pl.BlockSpec((1,), lambda i: (0,)),
        ],
        out_specs=pl.BlockSpec((MW * BPW, 1), lambda i: (i, 0)),
        out_shape=jax.ShapeDtypeStruct((B, 1), jnp.float32),
        interpret=interpret,
    )


def kernel(cast_idx, crew_idx, cast_table, crew_table, W1, b1, W2, b2):
    cidx = cast_idx.astype(jnp.int32).reshape(NW, BPW * L)
    kidx = crew_idx.astype(jnp.int32).reshape(NW, BPW * L)
    def _pack(tab, n):
        t = (tab * (1.0 / L)).astype(jnp.bfloat16)
        t = jnp.stack([t[:, :EMB // 2], t[:, EMB // 2:]], axis=-1)
        return lax.bitcast_convert_type(t, jnp.int32).reshape(-1)

    pooled = _make_pool_kernel()(
        _pack(cast_table, NUM_CAST), _pack(crew_table, NUM_CREW), cidx, kidx)
    return _make_mlp_call()(pooled, W1, b1, W2, b2)


# speedup vs baseline: 1.0019x; 1.0008x over previous
import functools

import jax
import jax.numpy as jnp
import numpy as np
from jax import lax
from jax.experimental import pallas as pl
from jax.experimental.pallas import tpu as pltpu
from jax.experimental.pallas import tpu_sc as plsc

B = 16384
L = 20
NUM_CAST = 1000
NUM_CREW = 1000
EMB = 32
HID = 128

NC = 2
NS = 16
LANES = 16
NW = NC * NS
BPW = B // NW

MW = 4


@functools.cache
def _make_pool_kernel(interpret=False):
    mesh = plsc.VectorSubcoreMesh(
        core_axis_name="c", subcore_axis_name="s",
        num_cores=NC, num_subcores=NS)

    @functools.partial(
        pl.kernel,
        out_type=jax.ShapeDtypeStruct((NW, BPW, 2 * EMB), jnp.float32),
        mesh=mesh,
        scratch_types=[
            pltpu.VMEM((NUM_CAST * EMB // 2,), jnp.int32),
            pltpu.VMEM((NUM_CREW * EMB // 2,), jnp.int32),
            pltpu.VMEM((BPW * L,), jnp.int32),
            pltpu.VMEM((BPW * L,), jnp.int32),
            pltpu.VMEM((BPW, 2 * EMB), jnp.float32),
        ],
        compiler_params=pltpu.CompilerParams(needs_layout_passes=False),
        interpret=interpret,
    )
    def pool(cast_tab_hbm, crew_tab_hbm, cidx_hbm, kidx_hbm, out_hbm,
             cast_v, crew_v, cidx_v, kidx_v, out_v):
        wid = lax.axis_index("s") * NC + lax.axis_index("c")
        pltpu.sync_copy(cast_tab_hbm, cast_v)
        pltpu.sync_copy(crew_tab_hbm, crew_v)
        pltpu.sync_copy(cidx_hbm.at[wid], cidx_v)
        pltpu.sync_copy(kidx_hbm.at[wid], kidx_v)

        @plsc.parallel_loop(0, BPW, unroll=8)
        def row(b):
            ib = b * L
            for idx_v, tab_v, off in ((cidx_v, cast_v, 0),
                                      (kidx_v, crew_v, EMB)):
                iv0 = idx_v[pl.ds(ib, LANES)] * (EMB // 2)
                iv1 = idx_v[pl.ds(ib + L - LANES, LANES)] * (EMB // 2)
                pa = [None, None, None, None]
                pb = [None, None, None, None]
                for l in range(L):
                    if l == 0:
                        r = iv0[0]
                    elif l < LANES:
                        r = iv0[l]
                    else:
                        r = iv1[l - (L - LANES)]
                    rw = plsc.bitcast(tab_v[pl.ds(r, LANES)], jnp.bfloat16)
                    a, c = plsc.unpack(rw, format=plsc.PackFormat.INTERLEAVED,
                                       preferred_element_type=jnp.float32)
                    k = l % 4
                    pa[k] = a if pa[k] is None else pa[k] + a
                    pb[k] = c if pb[k] is None else pb[k] + c
                ea = (pa[0] + pa[1]) + (pa[2] + pa[3])
                eb = (pb[0] + pb[1]) + (pb[2] + pb[3])
                out_v[b, pl.ds(off, LANES)] = ea
                out_v[b, pl.ds(off + LANES, LANES)] = eb

        pltpu.sync_copy(out_v, out_hbm.at[wid])

    return pool


def _mlp_body(x_ref, w1_ref, b1_ref, w2_ref, b2_ref, o_ref):
    for w in range(MW):
        x = x_ref[w]
        h = lax.dot_general(x, w1_ref[...], (((1,), (0,)), ((), ())),
                            preferred_element_type=jnp.float32)
        h = jnp.maximum(h + b1_ref[...][None, :], 0.0)
        o = lax.dot_general(h, w2_ref[...], (((1,), (0,)), ((), ())),
                            preferred_element_type=jnp.float32)
        o_ref[pl.ds(w * BPW, BPW), :] = o + b2_ref[...][None, :]


@functools.cache
def _make_mlp_call(interpret=False):
    return pl.pallas_call(
        _mlp_body,
        grid=(NW // MW,),
        in_specs=[
            pl.BlockSpec((MW, BPW, 2 * EMB), lambda i: (i, 0, 0)),
            pl.BlockSpec((2 * EMB, HID), lambda i: (0, 0)),
            pl.BlockSpec((HID,), lambda i: (0,)),
            pl.BlockSpec((HID, 1), lambda i: (0, 0)),
            pl.BlockSpec((1,), lambda i: (0,)),
        ],
        out_specs=pl.BlockSpec((MW * BPW, 1), lambda i: (i, 0)),
        out_shape=jax.ShapeDtypeStruct((B, 1), jnp.float32),
        interpret=interpret,
    )


def kernel(cast_idx, crew_idx, cast_table, crew_table, W1, b1, W2, b2):
    cidx = cast_idx.astype(jnp.int32).reshape(NW, BPW * L)
    kidx = crew_idx.astype(jnp.int32).reshape(NW, BPW * L)
    def _pack(tab, n):
        s = tab * (1.0 / L)
        lo = lax.bitcast_convert_type(
            s[:, :EMB // 2].astype(jnp.bfloat16), jnp.uint16)
        hi = lax.bitcast_convert_type(
            s[:, EMB // 2:].astype(jnp.bfloat16), jnp.uint16)
        w = lo.astype(jnp.uint32) | (hi.astype(jnp.uint32) << 16)
        return lax.bitcast_convert_type(w, jnp.int32).reshape(-1)

    pooled = _make_pool_kernel()(
        _pack(cast_table, NUM_CAST), _pack(crew_table, NUM_CREW), cidx, kidx)
    return _make_mlp_call()(pooled, W1, b1, W2, b2)


# speedup vs baseline: 1.0111x; 1.0092x over previous
import functools

import jax
import jax.numpy as jnp
import numpy as np
from jax import lax
from jax.experimental import pallas as pl
from jax.experimental.pallas import tpu as pltpu
from jax.experimental.pallas import tpu_sc as plsc

B = 16384
L = 20
NUM_CAST = 1000
NUM_CREW = 1000
EMB = 32
HID = 128

NC = 2
NS = 16
LANES = 16
NW = NC * NS
NSPLIT = 2
BC = B // NSPLIT
BPW = BC // NW

MW = 4


@functools.cache
def _make_pool_kernel(interpret=False):
    mesh = plsc.VectorSubcoreMesh(
        core_axis_name="c", subcore_axis_name="s",
        num_cores=NC, num_subcores=NS)

    @functools.partial(
        pl.kernel,
        out_type=jax.ShapeDtypeStruct((NW, BPW, 2 * EMB), jnp.float32),
        mesh=mesh,
        scratch_types=[
            pltpu.VMEM((NUM_CAST * EMB // 2,), jnp.int32),
            pltpu.VMEM((NUM_CREW * EMB // 2,), jnp.int32),
            pltpu.VMEM((BPW * L,), jnp.int32),
            pltpu.VMEM((BPW * L,), jnp.int32),
            pltpu.VMEM((BPW, 2 * EMB), jnp.float32),
        ],
        compiler_params=pltpu.CompilerParams(needs_layout_passes=False),
        interpret=interpret,
    )
    def pool(cast_tab_hbm, crew_tab_hbm, cidx_hbm, kidx_hbm, out_hbm,
             cast_v, crew_v, cidx_v, kidx_v, out_v):
        wid = lax.axis_index("s") * NC + lax.axis_index("c")
        pltpu.sync_copy(cast_tab_hbm, cast_v)
        pltpu.sync_copy(crew_tab_hbm, crew_v)
        pltpu.sync_copy(cidx_hbm.at[wid], cidx_v)
        pltpu.sync_copy(kidx_hbm.at[wid], kidx_v)

        @plsc.parallel_loop(0, BPW, unroll=8)
        def row(b):
            ib = b * L
            for idx_v, tab_v, off in ((cidx_v, cast_v, 0),
                                      (kidx_v, crew_v, EMB)):
                iv0 = idx_v[pl.ds(ib, LANES)] * (EMB // 2)
                iv1 = idx_v[pl.ds(ib + L - LANES, LANES)] * (EMB // 2)
                pa = [None, None, None, None]
                pb = [None, None, None, None]
                for l in range(L):
                    if l == 0:
                        r = iv0[0]
                    elif l < LANES:
                        r = iv0[l]
                    else:
                        r = iv1[l - (L - LANES)]
                    rw = plsc.bitcast(tab_v[pl.ds(r, LANES)], jnp.bfloat16)
                    a, c = plsc.unpack(rw, format=plsc.PackFormat.INTERLEAVED,
                                       preferred_element_type=jnp.float32)
                    k = l % 4
                    pa[k] = a if pa[k] is None else pa[k] + a
                    pb[k] = c if pb[k] is None else pb[k] + c
                ea = (pa[0] + pa[1]) + (pa[2] + pa[3])
                eb = (pb[0] + pb[1]) + (pb[2] + pb[3])
                out_v[b, pl.ds(off, LANES)] = ea
                out_v[b, pl.ds(off + LANES, LANES)] = eb

        pltpu.sync_copy(out_v, out_hbm.at[wid])

    return pool


def _mlp_body(x_ref, w1_ref, b1_ref, w2_ref, b2_ref, o_ref):
    for w in range(MW):
        x = x_ref[w]
        h = lax.dot_general(x, w1_ref[...], (((1,), (0,)), ((), ())),
                            preferred_element_type=jnp.float32)
        h = jnp.maximum(h + b1_ref[...][None, :], 0.0)
        o = lax.dot_general(h, w2_ref[...], (((1,), (0,)), ((), ())),
                            preferred_element_type=jnp.float32)
        o_ref[pl.ds(w * BPW, BPW), :] = o + b2_ref[...][None, :]


@functools.cache
def _make_mlp_call(interpret=False):
    return pl.pallas_call(
        _mlp_body,
        grid=(NW // MW,),
        in_specs=[
            pl.BlockSpec((MW, BPW, 2 * EMB), lambda i: (i, 0, 0)),
            pl.BlockSpec((2 * EMB, HID), lambda i: (0, 0)),
            pl.BlockSpec((HID,), lambda i: (0,)),
            pl.BlockSpec((HID, 1), lambda i: (0, 0)),
            pl.BlockSpec((1,), lambda i: (0,)),
        ],
        out_specs=pl.BlockSpec((MW * BPW, 1), lambda i: (i, 0)),
        out_shape=jax.ShapeDtypeStruct((BC, 1), jnp.float32),
        interpret=interpret,
    )


def kernel(cast_idx, crew_idx, cast_table, crew_table, W1, b1, W2, b2):
    def _pack(tab, n):
        s = tab * (1.0 / L)
        lo = lax.bitcast_convert_type(
            s[:, :EMB // 2].astype(jnp.bfloat16), jnp.uint16)
        hi = lax.bitcast_convert_type(
            s[:, EMB // 2:].astype(jnp.bfloat16), jnp.uint16)
        w = lo.astype(jnp.uint32) | (hi.astype(jnp.uint32) << 16)
        return lax.bitcast_convert_type(w, jnp.int32).reshape(-1)

    ct = _pack(cast_table, NUM_CAST)
    kt = _pack(crew_table, NUM_CREW)
    outs = []
    for g in range(NSPLIT):
        ci = lax.slice_in_dim(cast_idx, g * BC, (g + 1) * BC, axis=0)
        ki = lax.slice_in_dim(crew_idx, g * BC, (g + 1) * BC, axis=0)
        cidx = ci.astype(jnp.int32).reshape(NW, BPW * L)
        kidx = ki.astype(jnp.int32).reshape(NW, BPW * L)
        pooled = _make_pool_kernel()(ct, kt, cidx, kidx)
        outs.append(_make_mlp_call()(pooled, W1, b1, W2, b2))
    return jnp.concatenate(outs, axis=0)
